# baseline (speedup 1.0000x reference)
import functools

import jax
import jax.numpy as jnp
from jax import lax
from jax.experimental import pallas as pl
from jax.experimental.pallas import tpu as pltpu
from jax.experimental.pallas import tpu_sc as plsc

_NC = 2
_NS = 16
_NW = _NC * _NS
_C = 128


@functools.lru_cache(maxsize=None)
def _make_gather(N, V, D):
    n_per_w = N // _NW
    n_chunks = n_per_w // _C
    mesh = plsc.VectorSubcoreMesh(core_axis_name="c", subcore_axis_name="s")

    @functools.partial(
        pl.kernel,
        out_type=jax.ShapeDtypeStruct((N, D), jnp.float32),
        mesh=mesh,
        scratch_types=[
            pltpu.VMEM((n_chunks, _C), jnp.int32),
            pltpu.VMEM((_C, D), jnp.float32),
            pltpu.SemaphoreType.DMA,
        ],
        compiler_params=pltpu.CompilerParams(use_tc_tiling_on_sc=False),
    )
    def gather_kernel(idx_hbm, table_hbm, out_hbm, idx_v, rows_v, sem):
        wid = lax.axis_index("s") * _NC + lax.axis_index("c")
        base = wid * n_per_w
        pltpu.sync_copy(idx_hbm.at[wid], idx_v)

        @pl.loop(0, n_chunks)
        def _(i):
            pltpu.async_copy(table_hbm.at[idx_v.at[i]], rows_v, sem).wait()
            pltpu.sync_copy(rows_v, out_hbm.at[pl.ds(base + i * _C, _C)])

    return gather_kernel


def kernel(x, table):
    B, S = x.shape
    V, D = table.shape
    N = B * S
    idx = x.reshape(_NW, (N // _NW) // _C, _C).astype(jnp.int32)
    out = _make_gather(N, V, D)(idx, table)
    return out.reshape(B, S, D)


# speedup vs baseline: 1.1202x; 1.1202x over previous
import functools

import jax
import jax.numpy as jnp
from jax import lax
from jax.experimental import pallas as pl
from jax.experimental.pallas import tpu as pltpu
from jax.experimental.pallas import tpu_sc as plsc

_NC = 2
_NS = 16
_NW = _NC * _NS
_C = 128


_R = 4


@functools.lru_cache(maxsize=None)
def _make_gather(N, V, D):
    n_per_w = N // _NW
    n_chunks = n_per_w // _C
    mesh = plsc.VectorSubcoreMesh(core_axis_name="c", subcore_axis_name="s")

    @functools.partial(
        pl.kernel,
        out_type=jax.ShapeDtypeStruct((N, D), jnp.float32),
        mesh=mesh,
        scratch_types=[
            pltpu.VMEM((n_chunks, _C), jnp.int32),
            pltpu.VMEM((_R, _C, D), jnp.float32),
        ]
        + [pltpu.SemaphoreType.DMA] * (2 * _R),
        compiler_params=pltpu.CompilerParams(use_tc_tiling_on_sc=False),
    )
    def gather_kernel(idx_hbm, table_hbm, out_hbm, idx_v, rows_v, *sems):
        gsem, osem = sems[:_R], sems[_R:]
        wid = lax.axis_index("s") * _NC + lax.axis_index("c")
        base = wid * n_per_w
        pltpu.sync_copy(idx_hbm.at[wid], idx_v)

        for b in range(_R):
            pltpu.async_copy(table_hbm.at[idx_v.at[b]], rows_v.at[b], gsem[b])

        @pl.loop(0, n_chunks, step=_R)
        def _(i0):
            for b in range(_R):
                i = i0 + b
                pltpu.make_async_copy(
                    table_hbm.at[idx_v.at[i]], rows_v.at[b], gsem[b]
                ).wait()
                wdesc = pltpu.async_copy(
                    rows_v.at[b], out_hbm.at[pl.ds(base + i * _C, _C)], osem[b]
                )

                @pl.when(i + _R < n_chunks)
                def _():
                    wdesc.wait()
                    pltpu.async_copy(
                        table_hbm.at[idx_v.at[i + _R]], rows_v.at[b], gsem[b]
                    )

        for b in range(_R):
            pltpu.make_async_copy(
                rows_v.at[b], out_hbm.at[pl.ds(base, _C)], osem[b]
            ).wait()

    return gather_kernel


def kernel(x, table):
    B, S = x.shape
    V, D = table.shape
    N = B * S
    idx = x.reshape(_NW, (N // _NW) // _C, _C).astype(jnp.int32)
    out = _make_gather(N, V, D)(idx, table)
    return out.reshape(B, S, D)
